# baseline (speedup 1.0000x reference)
import jax
import jax.numpy as jnp
from jax.experimental import pallas as pl

VOCAB = 1000
ROWS = 1024
SEQ = 50
BLOCK_R = 64


def _onehot_body(ids_ref, mask_ref, out_ref):
    ids = ids_ref[...]
    mask = mask_ref[...]
    iota = jax.lax.broadcasted_iota(jnp.int32, (BLOCK_R, SEQ, VOCAB), 2)
    out_ref[...] = jnp.where(iota == ids[:, :, None], mask[:, :, None], 0.0)


def kernel(input_ids, attention_mask):
    ids = input_ids.astype(jnp.int32)
    mask = attention_mask.astype(jnp.float32)
    grid = (ROWS // BLOCK_R,)
    return pl.pallas_call(
        _onehot_body,
        grid=grid,
        in_specs=[
            pl.BlockSpec((BLOCK_R, SEQ), lambda i: (i, 0)),
            pl.BlockSpec((BLOCK_R, SEQ), lambda i: (i, 0)),
        ],
        out_specs=pl.BlockSpec((BLOCK_R, SEQ, VOCAB), lambda i: (i, 0, 0)),
        out_shape=jax.ShapeDtypeStruct((ROWS, SEQ, VOCAB), jnp.float32),
    )(ids, mask)


# speedup vs baseline: 1.0056x; 1.0056x over previous
import jax
import jax.numpy as jnp
from jax.experimental import pallas as pl
from jax.experimental.pallas import tpu as pltpu

VOCAB = 1000
ROWS = 1024
SEQ = 50
BLOCK_R = 16
NBUF = 8
GRID = ROWS // BLOCK_R


def _onehot_body(ids_ref, mask_ref, out_hbm, scratch, sems):
    i = pl.program_id(0)
    slot = jax.lax.rem(i, NBUF)

    @pl.when(i >= NBUF)
    def _wait_prev():
        j = i - NBUF
        pltpu.make_async_copy(
            scratch.at[jax.lax.rem(j, NBUF)],
            out_hbm.at[pl.ds(j * BLOCK_R, BLOCK_R)],
            sems.at[jax.lax.rem(j, NBUF)],
        ).wait()

    ids = ids_ref[...]
    mask = mask_ref[...]
    iota = jax.lax.broadcasted_iota(jnp.int32, (BLOCK_R, SEQ, VOCAB), 2)
    scratch[slot] = jnp.where(iota == ids[:, :, None], mask[:, :, None], 0.0)

    pltpu.make_async_copy(
        scratch.at[slot],
        out_hbm.at[pl.ds(i * BLOCK_R, BLOCK_R)],
        sems.at[slot],
    ).start()

    @pl.when(i == GRID - 1)
    def _drain():
        for k in range(NBUF):
            j = i - (NBUF - 1) + k
            pltpu.make_async_copy(
                scratch.at[jax.lax.rem(j, NBUF)],
                out_hbm.at[pl.ds(j * BLOCK_R, BLOCK_R)],
                sems.at[jax.lax.rem(j, NBUF)],
            ).wait()


def kernel(input_ids, attention_mask):
    ids = input_ids.astype(jnp.int32)
    mask = attention_mask.astype(jnp.float32)
    return pl.pallas_call(
        _onehot_body,
        grid=(GRID,),
        in_specs=[
            pl.BlockSpec((BLOCK_R, SEQ), lambda i: (i, 0)),
            pl.BlockSpec((BLOCK_R, SEQ), lambda i: (i, 0)),
        ],
        out_specs=pl.BlockSpec(memory_space=pltpu.HBM),
        out_shape=jax.ShapeDtypeStruct((ROWS, SEQ, VOCAB), jnp.float32),
        scratch_shapes=[
            pltpu.VMEM((NBUF, BLOCK_R, SEQ, VOCAB), jnp.float32),
            pltpu.SemaphoreType.DMA((NBUF,)),
        ],
    )(ids, mask)
